# Initial kernel scaffold; baseline (speedup 1.0000x reference)
import jax
import jax.numpy as jnp
from jax.experimental import pallas as pl


def kernel(flat, cu_seqlens):
    raise NotImplementedError("write your pallas kernel here")


# speedup vs baseline: 1.7587x; 1.7587x over previous
import functools

import jax
import jax.numpy as jnp
from jax import lax
from jax.experimental import pallas as pl
from jax.experimental.pallas import tpu as pltpu
from jax.experimental.pallas import tpu_sc as plsc

B = 8
L = 2048
D = 512
TOTAL = B * L // 2
NROWS = B * L

NC = 2
NS = 16
NW = NC * NS
LANES = 16

TPW = TOTAL // NW
CH = 64
NCH = TPW // CH

_mesh = plsc.VectorSubcoreMesh(core_axis_name="c", subcore_axis_name="s")


@functools.partial(
    pl.kernel,
    out_type=jax.ShapeDtypeStruct((NROWS, D), jnp.float32),
    mesh=_mesh,
    scratch_types=[
        pltpu.VMEM((16,), jnp.int32),
        pltpu.VMEM((NCH, CH), jnp.int32),
        pltpu.VMEM((NCH, CH), jnp.int32),
        pltpu.VMEM((2, CH, D), jnp.float32),
        pltpu.VMEM((CH, D), jnp.float32),
        pltpu.SemaphoreType.DMA,
    ],
    compiler_params=pltpu.CompilerParams(needs_layout_passes=False),
)
def _to_dense(flat_hbm, cu_hbm, zeros_hbm, out_hbm,
              cu_v, didx, pidx, dbuf, zbuf, sem):
    wid = lax.axis_index("s") * NC + lax.axis_index("c")
    base = wid * TPW

    pltpu.sync_copy(cu_hbm, cu_v)
    pltpu.sync_copy(zeros_hbm, zbuf)

    iota = lax.iota(jnp.int32, LANES)
    cub = [plsc.load_gather(cu_v, [jnp.full((LANES,), k, jnp.int32)])
           for k in range(1, B + 1)]

    for c in range(NCH):
        for j in range(CH // LANES):
            iv = (base + c * CH + j * LANES) + iota
            r = jnp.zeros((LANES,), jnp.int32)
            for k in range(B):
                r = r + (iv >= cub[k]).astype(jnp.int32)
            didx[c, pl.ds(j * LANES, LANES)] = (
                r * L + iv - plsc.load_gather(cu_v, [r]))
            b = jnp.zeros((LANES,), jnp.int32)
            for k in range(B):
                b = b + (iv >= ((k + 1) * L - cub[k])).astype(jnp.int32)
            pidx[c, pl.ds(j * LANES, LANES)] = (
                plsc.load_gather(cu_v, [b + 1]) + iv)

    for c in range(NCH):
        pltpu.sync_copy(zbuf, out_hbm.at[pidx.at[c]])
    for c in range(NCH):
        pltpu.sync_copy(flat_hbm.at[pl.ds(base + c * CH, CH)], dbuf.at[c % 2])
        pltpu.sync_copy(dbuf.at[c % 2], out_hbm.at[didx.at[c]])


def kernel(flat, cu_seqlens):
    cu_pad = jnp.zeros((16,), jnp.int32).at[:B + 1].set(
        cu_seqlens.astype(jnp.int32))
    zeros = jnp.zeros((CH, D), jnp.float32)
    return _to_dense(flat, cu_pad, zeros).reshape(B, L, D)


# speedup vs baseline: 1.9487x; 1.1080x over previous
import functools

import jax
import jax.numpy as jnp
from jax import lax
from jax.experimental import pallas as pl
from jax.experimental.pallas import tpu as pltpu
from jax.experimental.pallas import tpu_sc as plsc

B = 8
L = 2048
D = 512
TOTAL = B * L // 2
NROWS = B * L

NC = 2
NS = 16
NW = NC * NS
LANES = 16

TPW = TOTAL // NW
CH = 64
NCH = TPW // CH

_mesh = plsc.VectorSubcoreMesh(core_axis_name="c", subcore_axis_name="s")


@functools.partial(
    pl.kernel,
    out_type=jax.ShapeDtypeStruct((NROWS, D), jnp.float32),
    mesh=_mesh,
    scratch_types=[
        pltpu.VMEM((16,), jnp.int32),
        pltpu.VMEM((NCH, CH), jnp.int32),
        pltpu.VMEM((NCH, CH), jnp.int32),
        pltpu.VMEM((2, CH, D), jnp.float32),
        pltpu.VMEM((CH, D), jnp.float32),
        pltpu.SemaphoreType.DMA,
        pltpu.SemaphoreType.DMA,
        pltpu.SemaphoreType.DMA,
        pltpu.SemaphoreType.DMA,
        pltpu.SemaphoreType.DMA,
        pltpu.SemaphoreType.DMA,
        pltpu.SemaphoreType.DMA,
    ],
    compiler_params=pltpu.CompilerParams(needs_layout_passes=False),
)
def _to_dense(flat_hbm, cu_hbm, zeros_hbm, out_hbm,
              cu_v, didx, pidx, dbuf, zbuf,
              sem_cu, sem_z, sem_in0, sem_in1, sem_sc0, sem_sc1, sem_p):
    wid = lax.axis_index("s") * NC + lax.axis_index("c")
    base = wid * TPW
    sem_in = (sem_in0, sem_in1)
    sem_sc = (sem_sc0, sem_sc1)

    def load(c):
        return pltpu.async_copy(
            flat_hbm.at[pl.ds(base + c * CH, CH)], dbuf.at[c % 2],
            sem_in[c % 2])

    def scatter(c):
        return pltpu.async_copy(
            dbuf.at[c % 2], out_hbm.at[didx.at[c]], sem_sc[c % 2])

    cp_cu = pltpu.async_copy(cu_hbm, cu_v, sem_cu)
    cp_z = pltpu.async_copy(zeros_hbm, zbuf, sem_z)
    lds = [load(0), load(1)]
    cp_cu.wait()

    iota = lax.iota(jnp.int32, LANES)
    cub = [plsc.load_gather(cu_v, [jnp.full((LANES,), k, jnp.int32)])
           for k in range(1, B + 1)]

    for c in range(NCH):
        for j in range(CH // LANES):
            iv = (base + c * CH + j * LANES) + iota
            r = jnp.zeros((LANES,), jnp.int32)
            for k in range(B):
                r = r + (iv >= cub[k]).astype(jnp.int32)
            didx[c, pl.ds(j * LANES, LANES)] = (
                r * L + iv - plsc.load_gather(cu_v, [r]))
            b = jnp.zeros((LANES,), jnp.int32)
            for k in range(B):
                b = b + (iv >= ((k + 1) * L - cub[k])).astype(jnp.int32)
            pidx[c, pl.ds(j * LANES, LANES)] = (
                plsc.load_gather(cu_v, [b + 1]) + iv)

    cp_z.wait()
    pads = [pltpu.async_copy(zbuf, out_hbm.at[pidx.at[c]], sem_p)
            for c in range(NCH)]

    scs = {}
    lds[0].wait()
    scs[0] = scatter(0)
    lds[1].wait()
    scs[1] = scatter(1)
    scs[0].wait()
    ld2 = load(2)
    scs[1].wait()
    ld3 = load(3)
    ld2.wait()
    scs[2] = scatter(2)
    ld3.wait()
    scs[3] = scatter(3)
    scs[2].wait()
    scs[3].wait()
    for p in pads:
        p.wait()


def kernel(flat, cu_seqlens):
    cu_pad = jnp.zeros((16,), jnp.int32).at[:B + 1].set(
        cu_seqlens.astype(jnp.int32))
    zeros = jnp.zeros((CH, D), jnp.float32)
    return _to_dense(flat, cu_pad, zeros).reshape(B, L, D)
